# Initial kernel scaffold; baseline (speedup 1.0000x reference)
import jax
import jax.numpy as jnp
from jax.experimental import pallas as pl


def kernel(flat_tokens, cu_seqlens, table):
    raise NotImplementedError("write your pallas kernel here")


# speedup vs baseline: 1.3818x; 1.3818x over previous
import functools

import jax
import jax.numpy as jnp
from jax import lax
from jax.experimental import pallas as pl
from jax.experimental.pallas import tpu as pltpu
from jax.experimental.pallas import tpu_sc as plsc

VOCAB = 100000
EMB = 256
TOTAL = 16384

_NC = 2
_NS = 16
_NW = _NC * _NS
_B_PER_W = TOTAL // _NW
_CHUNK = 128
_N_CHUNKS = _B_PER_W // _CHUNK

_mesh = plsc.VectorSubcoreMesh(core_axis_name="c", subcore_axis_name="s")


@functools.partial(
    pl.kernel,
    mesh=_mesh,
    out_type=jax.ShapeDtypeStruct((TOTAL, EMB), jnp.float32),
    scratch_types=[
        pltpu.VMEM((_B_PER_W,), jnp.int32),
        pltpu.VMEM((_CHUNK, EMB), jnp.float32),
        pltpu.VMEM((_CHUNK, EMB), jnp.float32),
        pltpu.SemaphoreType.DMA,
        pltpu.SemaphoreType.DMA,
    ],
)
def _gather_kernel(tokens_hbm, table_hbm, out_hbm, idx_v, rows0, rows1, sem0, sem1):
    wid = lax.axis_index("s") * _NC + lax.axis_index("c")
    base = wid * _B_PER_W
    pltpu.sync_copy(tokens_hbm.at[pl.ds(base, _B_PER_W)], idx_v)

    bufs = (rows0, rows1)
    sems = (sem0, sem1)
    copies = [None, None]
    for j in range(_N_CHUNKS):
        b = j % 2
        copies[b] = pltpu.async_copy(
            table_hbm.at[idx_v.at[pl.ds(j * _CHUNK, _CHUNK)]], bufs[b], sems[b]
        )
        prev = 1 - b
        if copies[prev] is not None:
            copies[prev].wait()
            pltpu.sync_copy(
                bufs[prev], out_hbm.at[pl.ds(base + (j - 1) * _CHUNK, _CHUNK)]
            )
    last = (_N_CHUNKS - 1) % 2
    copies[last].wait()
    pltpu.sync_copy(
        bufs[last], out_hbm.at[pl.ds(base + (_N_CHUNKS - 1) * _CHUNK, _CHUNK)]
    )


def kernel(flat_tokens, cu_seqlens, table):
    del cu_seqlens
    return _gather_kernel(flat_tokens, table)


# speedup vs baseline: 1.4172x; 1.0256x over previous
import functools

import jax
import jax.numpy as jnp
from jax import lax
from jax.experimental import pallas as pl
from jax.experimental.pallas import tpu as pltpu
from jax.experimental.pallas import tpu_sc as plsc

VOCAB = 100000
EMB = 256
TOTAL = 16384

_NC = 2
_NS = 16
_NW = _NC * _NS
_B_PER_W = TOTAL // _NW
_CHUNK = 128
_N_CHUNKS = _B_PER_W // _CHUNK
_NBUF = 3

_mesh = plsc.VectorSubcoreMesh(core_axis_name="c", subcore_axis_name="s")


@functools.partial(
    pl.kernel,
    mesh=_mesh,
    out_type=jax.ShapeDtypeStruct((TOTAL, EMB), jnp.float32),
    scratch_types=[
        pltpu.VMEM((_B_PER_W,), jnp.int32),
    ]
    + [pltpu.VMEM((_CHUNK, EMB), jnp.float32) for _ in range(_NBUF)]
    + [pltpu.SemaphoreType.DMA for _ in range(2 * _NBUF)],
)
def _gather_kernel(tokens_hbm, table_hbm, out_hbm, idx_v, *bufs_sems):
    bufs = bufs_sems[:_NBUF]
    gsems = bufs_sems[_NBUF : 2 * _NBUF]
    wsems = bufs_sems[2 * _NBUF :]
    wid = lax.axis_index("s") * _NC + lax.axis_index("c")
    base = wid * _B_PER_W
    pltpu.sync_copy(tokens_hbm.at[pl.ds(base, _B_PER_W)], idx_v)

    gcp = [None] * _NBUF
    wcp = [None] * _NBUF
    for j in range(_N_CHUNKS):
        b = j % _NBUF
        if j >= _NBUF:
            wcp[b].wait()
        gcp[b] = pltpu.async_copy(
            table_hbm.at[idx_v.at[pl.ds(j * _CHUNK, _CHUNK)]], bufs[b], gsems[b]
        )
        d = j - (_NBUF - 1)
        if d >= 0:
            db = d % _NBUF
            gcp[db].wait()
            wcp[db] = pltpu.async_copy(
                bufs[db], out_hbm.at[pl.ds(base + d * _CHUNK, _CHUNK)], wsems[db]
            )
    for d in range(max(0, _N_CHUNKS - (_NBUF - 1)), _N_CHUNKS):
        db = d % _NBUF
        gcp[db].wait()
        wcp[db] = pltpu.async_copy(
            bufs[db], out_hbm.at[pl.ds(base + d * _CHUNK, _CHUNK)], wsems[db]
        )
    for d in range(max(0, _N_CHUNKS - _NBUF), _N_CHUNKS):
        wcp[d % _NBUF].wait()


def kernel(flat_tokens, cu_seqlens, table):
    del cu_seqlens
    return _gather_kernel(flat_tokens, table)


# speedup vs baseline: 1.4424x; 1.0178x over previous
import functools

import jax
import jax.numpy as jnp
from jax import lax
from jax.experimental import pallas as pl
from jax.experimental.pallas import tpu as pltpu
from jax.experimental.pallas import tpu_sc as plsc

VOCAB = 100000
EMB = 256
TOTAL = 16384

_NC = 2
_NS = 16
_NW = _NC * _NS
_B_PER_W = TOTAL // _NW
_CHUNK = 64
_N_CHUNKS = _B_PER_W // _CHUNK
_NBUF = 6

_mesh = plsc.VectorSubcoreMesh(core_axis_name="c", subcore_axis_name="s")


@functools.partial(
    pl.kernel,
    mesh=_mesh,
    out_type=jax.ShapeDtypeStruct((TOTAL, EMB), jnp.float32),
    scratch_types=[
        pltpu.VMEM((_B_PER_W,), jnp.int32),
    ]
    + [pltpu.VMEM((_CHUNK, EMB), jnp.float32) for _ in range(_NBUF)]
    + [pltpu.SemaphoreType.DMA for _ in range(2 * _NBUF)],
)
def _gather_kernel(tokens_hbm, table_hbm, out_hbm, idx_v, *bufs_sems):
    bufs = bufs_sems[:_NBUF]
    gsems = bufs_sems[_NBUF : 2 * _NBUF]
    wsems = bufs_sems[2 * _NBUF :]
    wid = lax.axis_index("s") * _NC + lax.axis_index("c")
    base = wid * _B_PER_W
    pltpu.sync_copy(tokens_hbm.at[pl.ds(base, _B_PER_W)], idx_v)

    gcp = [None] * _NBUF
    wcp = [None] * _NBUF
    for j in range(_N_CHUNKS):
        b = j % _NBUF
        if j >= _NBUF:
            wcp[b].wait()
        gcp[b] = pltpu.async_copy(
            table_hbm.at[idx_v.at[pl.ds(j * _CHUNK, _CHUNK)]], bufs[b], gsems[b]
        )
        d = j - (_NBUF - 1)
        if d >= 0:
            db = d % _NBUF
            gcp[db].wait()
            wcp[db] = pltpu.async_copy(
                bufs[db], out_hbm.at[pl.ds(base + d * _CHUNK, _CHUNK)], wsems[db]
            )
    for d in range(max(0, _N_CHUNKS - (_NBUF - 1)), _N_CHUNKS):
        db = d % _NBUF
        gcp[db].wait()
        wcp[db] = pltpu.async_copy(
            bufs[db], out_hbm.at[pl.ds(base + d * _CHUNK, _CHUNK)], wsems[db]
        )
    for d in range(max(0, _N_CHUNKS - _NBUF), _N_CHUNKS):
        wcp[d % _NBUF].wait()


def kernel(flat_tokens, cu_seqlens, table):
    del cu_seqlens
    return _gather_kernel(flat_tokens, table)
